# baseline (speedup 1.0000x reference)
import functools

import jax
import jax.numpy as jnp
from jax import lax
from jax.experimental import pallas as pl
from jax.experimental.pallas import tpu as pltpu
from jax.experimental.pallas import tpu_sc as plsc

_NC = 2
_NS = 16
_L = 16
_NW = _NC * _NS

_GCHUNK = 128

_MESH = dict(core_axis_name="c", subcore_axis_name="s")
_SKIP_T = False
_PARAMS = pltpu.CompilerParams(needs_layout_passes=False)


def _transpose_block(in_ref, out_ref, n_rows, lanes, evecs):
    cols = [evecs[g % 4] + 64 * (g // 4) for g in range(8)]

    @plsc.parallel_loop(0, n_rows, unroll=4)
    def _(r):
        c0 = jnp.full((_L,), 2 * r, jnp.int32)
        c1 = c0 + 1
        vs = [plsc.load_gather(in_ref, [evecs[g % 4], c0 if g < 4 else c1])
              for g in range(8)]
        rsp = jnp.full((_L,), r, jnp.int32)
        for g in range(8):
            plsc.store_scatter(out_ref, [rsp, cols[g]], vs[g])


def _make_relayout_kernel(V, E):
    W = 2 * E
    n_full = V // W
    tail = V - n_full * W
    r_out = V // 2

    mesh = plsc.VectorSubcoreMesh(**_MESH)

    @functools.partial(
        pl.kernel,
        mesh=mesh,
        compiler_params=_PARAMS,
        out_type=(jax.ShapeDtypeStruct((r_out, W), jnp.float32),
                  jax.ShapeDtypeStruct((r_out, W), jnp.float32)),
        scratch_types=[
            pltpu.VMEM((E, W), jnp.float32), pltpu.VMEM((E, W), jnp.float32),
            pltpu.VMEM((E, W), jnp.float32), pltpu.VMEM((E, W), jnp.float32),
            pltpu.VMEM((E, W), jnp.float32), pltpu.VMEM((E, W), jnp.float32),
            pltpu.VMEM((E, W), jnp.float32), pltpu.VMEM((E, W), jnp.float32),
            pltpu.VMEM((E, E), jnp.float32),
            pltpu.VMEM((E // 2, W), jnp.float32),
            pltpu.SemaphoreType.DMA, pltpu.SemaphoreType.DMA,
            pltpu.SemaphoreType.DMA, pltpu.SemaphoreType.DMA,
            pltpu.SemaphoreType.DMA, pltpu.SemaphoreType.DMA,
            pltpu.SemaphoreType.DMA, pltpu.SemaphoreType.DMA,
        ],
    )
    def k1(tt_hbm, tc_hbm, st_hbm, sc_hbm,
           it0, it1, ic0, ic1, ot0, ot1, oc0, oc1, itail, otail,
           sit0, sit1, sic0, sic1, sot0, sot1, soc0, soc1):
        in_t, in_c = (it0, it1), (ic0, ic1)
        out_t, out_c = (ot0, ot1), (oc0, oc1)
        sin_t, sin_c = (sit0, sit1), (sic0, sic1)
        sout_t, sout_c = (sot0, sot1), (soc0, soc1)

        wid = lax.axis_index("s") * _NC + lax.axis_index("c")
        lanes = lax.iota(jnp.int32, _L)
        evecs = [lanes + 16 * q for q in range(4)]

        def start_in(k, b):
            blk = wid + _NW * k

            @pl.when(blk < n_full)
            def _():
                col = blk * W
                pltpu.async_copy(tt_hbm.at[:, pl.ds(col, W)], in_t[b],
                                 sin_t[b])
                pltpu.async_copy(tc_hbm.at[:, pl.ds(col, W)], in_c[b],
                                 sin_c[b])

        start_in(0, 0)

        def g_body(g, _):
            for b in (0, 1):
                k = 2 * g + b
                start_in(k + 1, 1 - b)
                blk = wid + _NW * k

                @pl.when(blk < n_full)
                def _():
                    col = blk * W
                    row = blk * E
                    pltpu.make_async_copy(
                        tt_hbm.at[:, pl.ds(col, W)], in_t[b],
                        sin_t[b]).wait()
                    pltpu.make_async_copy(
                        tc_hbm.at[:, pl.ds(col, W)], in_c[b],
                        sin_c[b]).wait()

                    @pl.when(k >= 2)
                    def _():
                        pltpu.make_async_copy(
                            out_t[b], st_hbm.at[pl.ds(row, E), :],
                            sout_t[b]).wait()
                        pltpu.make_async_copy(
                            out_c[b], sc_hbm.at[pl.ds(row, E), :],
                            sout_c[b]).wait()

                    if _SKIP_T:
                        pass
                    else:
                        _transpose_block(in_t[b], out_t[b], E, lanes, evecs)
                        _transpose_block(in_c[b], out_c[b], E, lanes, evecs)
                    pltpu.async_copy(out_t[b], st_hbm.at[pl.ds(row, E), :],
                                     sout_t[b])
                    pltpu.async_copy(out_c[b], sc_hbm.at[pl.ds(row, E), :],
                                     sout_c[b])
            return 0

        n_iter = (n_full + _NW - 1) // _NW + 1
        lax.fori_loop(0, (n_iter + 1) // 2, g_body, 0)
        for b in (0, 1):
            pltpu.make_async_copy(out_t[b], st_hbm.at[pl.ds(0, E), :],
                                  sout_t[b]).wait()
            pltpu.make_async_copy(out_c[b], sc_hbm.at[pl.ds(0, E), :],
                                  sout_c[b]).wait()

        if tail:
            @pl.when(wid < 2)
            def _():
                @pl.when(wid == 0)
                def _():
                    pltpu.sync_copy(tt_hbm.at[:, pl.ds(n_full * W, tail)],
                                    itail)
                    _transpose_block(itail, otail, tail // 2, lanes, evecs)
                    pltpu.sync_copy(
                        otail, st_hbm.at[pl.ds(n_full * E, tail // 2), :])

                @pl.when(wid == 1)
                def _():
                    pltpu.sync_copy(tc_hbm.at[:, pl.ds(n_full * W, tail)],
                                    itail)
                    _transpose_block(itail, otail, tail // 2, lanes, evecs)
                    pltpu.sync_copy(
                        otail, sc_hbm.at[pl.ds(n_full * E, tail // 2), :])

    return k1


def _make_dots_kernel(B, C, E, Cb):
    n_chunks = (B // _NW) // Cb
    assert Cb % _L == 0 and (B // _NW) % Cb == 0
    assert (Cb * C) % _GCHUNK == 0
    n_cgather = (Cb * C) // _GCHUNK
    W = 2 * E

    mesh = plsc.VectorSubcoreMesh(**_MESH)

    @functools.partial(
        pl.kernel,
        mesh=mesh,
        compiler_params=_PARAMS,
        out_type=jax.ShapeDtypeStruct((B * C,), jnp.float32),
        scratch_types=[
            pltpu.VMEM((Cb,), jnp.int32),
            pltpu.VMEM((Cb,), jnp.int32),
            pltpu.VMEM((Cb * C,), jnp.int32),
            pltpu.VMEM((Cb * C,), jnp.int32),
            pltpu.VMEM((Cb, W), jnp.float32),
            pltpu.VMEM((Cb * C, W), jnp.float32),
            pltpu.VMEM((Cb * C,), jnp.float32),
            pltpu.SemaphoreType.DMA,
        ],
    )
    def sc_k(thi_hbm, toff_hbm, chi_hbm, coff_hbm, ttab_hbm, ctab_hbm,
             out_hbm, tgt_idx, tgt_off, ctx_idx, ctx_off, te_rows, ce_rows,
             out_v, sem):
        wid = lax.axis_index("s") * _NC + lax.axis_index("c")
        lanes = lax.iota(jnp.int32, _L)

        for i in range(n_chunks):
            base_b = wid * (B // _NW) + i * Cb
            pltpu.sync_copy(thi_hbm.at[pl.ds(base_b, Cb)], tgt_idx)
            pltpu.sync_copy(toff_hbm.at[pl.ds(base_b, Cb)], tgt_off)
            pltpu.sync_copy(chi_hbm.at[pl.ds(base_b * C, Cb * C)], ctx_idx)
            pltpu.sync_copy(coff_hbm.at[pl.ds(base_b * C, Cb * C)], ctx_off)

            cps = [pltpu.async_copy(ttab_hbm.at[tgt_idx], te_rows, sem)]
            for j in range(n_cgather):
                cps.append(pltpu.async_copy(
                    ctab_hbm.at[ctx_idx.at[pl.ds(j * _GCHUNK, _GCHUNK)]],
                    ce_rows.at[pl.ds(j * _GCHUNK, _GCHUNK)], sem))
            for cp in cps:
                cp.wait()

            @plsc.parallel_loop(0, Cb // _L)
            def _(g):
                b_ids = g * _L + lanes
                flat0 = b_ids * C
                toff = plsc.load_gather(tgt_off, [b_ids])
                coffs = [plsc.load_gather(ctx_off, [flat0 + c])
                         for c in range(C)]
                zeros = tuple(jnp.zeros((_L,), jnp.float32)
                              for _ in range(C))

                @plsc.parallel_loop(0, E, unroll=2, carry=zeros)
                def accs(e, accs):
                    ev = jnp.full((_L,), e, jnp.int32)
                    tv = plsc.load_gather(te_rows, [b_ids, toff + ev])
                    return tuple(
                        accs[c] + tv * plsc.load_gather(
                            ce_rows, [flat0 + c, coffs[c] + ev])
                        for c in range(C))

                for c in range(C):
                    plsc.store_scatter(out_v, [flat0 + c], accs[c])
            pltpu.sync_copy(out_v, out_hbm.at[pl.ds(base_b * C, Cb * C)])

    return sc_k


def kernel(target, context, target_table, context_table):
    B, C = context.shape
    V, E = target_table.shape
    ctx_flat = context.reshape(-1)
    t_hi = lax.shift_right_logical(target, 1)
    t_off = (target & 1) * E
    c_hi = lax.shift_right_logical(ctx_flat, 1)
    c_off = (ctx_flat & 1) * E
    k1 = _make_relayout_kernel(V, E)
    stag_t, stag_c = k1(target_table.T, context_table.T)
    sc_k = _make_dots_kernel(B, C, E, Cb=128)
    out = sc_k(t_hi, t_off, c_hi, c_off, stag_t, stag_c)
    return out.reshape(B, C)


# speedup vs baseline: 2.4808x; 2.4808x over previous
import functools

import jax
import jax.numpy as jnp
from jax import lax
from jax.experimental import pallas as pl
from jax.experimental.pallas import tpu as pltpu
from jax.experimental.pallas import tpu_sc as plsc

_NC = 2
_NS = 16
_L = 16
_NW = _NC * _NS

_GCHUNK = 128

_MESH = dict(core_axis_name="c", subcore_axis_name="s")
_SKIP_T = False
_PARAMS = pltpu.CompilerParams(needs_layout_passes=False)


def _transpose_block(in_ref, out_ref, n_cols, lanes, evecs):
    n_j = n_cols // 16

    @plsc.parallel_loop(0, 16, unroll=2)
    def _(m):
        cm = (jnp.full((_L,), m, jnp.int32) + lanes) & 15
        cmh = cm >> 1
        podd = (cm & 1) << 6
        ps = [evecs[q] + podd for q in range(4)]
        for j in range(n_j):
            cj = cm + (16 * j)
            rj = cmh + (8 * j)
            for q in range(4):
                v = plsc.load_gather(in_ref, [evecs[q], cj])
                plsc.store_scatter(out_ref, [rj, ps[q]], v)


def _make_relayout_kernel(V, E):
    W = 2 * E
    n_full = V // W
    tail = V - n_full * W
    r_out = V // 2

    mesh = plsc.VectorSubcoreMesh(**_MESH)

    @functools.partial(
        pl.kernel,
        mesh=mesh,
        compiler_params=_PARAMS,
        out_type=(jax.ShapeDtypeStruct((r_out, W), jnp.float32),
                  jax.ShapeDtypeStruct((r_out, W), jnp.float32)),
        scratch_types=[
            pltpu.VMEM((E, W), jnp.float32), pltpu.VMEM((E, W), jnp.float32),
            pltpu.VMEM((E, W), jnp.float32), pltpu.VMEM((E, W), jnp.float32),
            pltpu.VMEM((E, W), jnp.float32), pltpu.VMEM((E, W), jnp.float32),
            pltpu.VMEM((E, W), jnp.float32), pltpu.VMEM((E, W), jnp.float32),
            pltpu.VMEM((E, E), jnp.float32),
            pltpu.VMEM((E // 2, W), jnp.float32),
            pltpu.SemaphoreType.DMA, pltpu.SemaphoreType.DMA,
            pltpu.SemaphoreType.DMA, pltpu.SemaphoreType.DMA,
            pltpu.SemaphoreType.DMA, pltpu.SemaphoreType.DMA,
            pltpu.SemaphoreType.DMA, pltpu.SemaphoreType.DMA,
        ],
    )
    def k1(tt_hbm, tc_hbm, st_hbm, sc_hbm,
           it0, it1, ic0, ic1, ot0, ot1, oc0, oc1, itail, otail,
           sit0, sit1, sic0, sic1, sot0, sot1, soc0, soc1):
        in_t, in_c = (it0, it1), (ic0, ic1)
        out_t, out_c = (ot0, ot1), (oc0, oc1)
        sin_t, sin_c = (sit0, sit1), (sic0, sic1)
        sout_t, sout_c = (sot0, sot1), (soc0, soc1)

        wid = lax.axis_index("s") * _NC + lax.axis_index("c")
        lanes = lax.iota(jnp.int32, _L)
        evecs = [lanes + 16 * q for q in range(4)]

        def start_in(k, b):
            blk = wid + _NW * k

            @pl.when(blk < n_full)
            def _():
                col = blk * W
                pltpu.async_copy(tt_hbm.at[:, pl.ds(col, W)], in_t[b],
                                 sin_t[b])
                pltpu.async_copy(tc_hbm.at[:, pl.ds(col, W)], in_c[b],
                                 sin_c[b])

        start_in(0, 0)

        def g_body(g, _):
            for b in (0, 1):
                k = 2 * g + b
                start_in(k + 1, 1 - b)
                blk = wid + _NW * k

                @pl.when(blk < n_full)
                def _():
                    col = blk * W
                    row = blk * E
                    pltpu.make_async_copy(
                        tt_hbm.at[:, pl.ds(col, W)], in_t[b],
                        sin_t[b]).wait()
                    pltpu.make_async_copy(
                        tc_hbm.at[:, pl.ds(col, W)], in_c[b],
                        sin_c[b]).wait()

                    @pl.when(k >= 2)
                    def _():
                        pltpu.make_async_copy(
                            out_t[b], st_hbm.at[pl.ds(row, E), :],
                            sout_t[b]).wait()
                        pltpu.make_async_copy(
                            out_c[b], sc_hbm.at[pl.ds(row, E), :],
                            sout_c[b]).wait()

                    if _SKIP_T:
                        pass
                    else:
                        _transpose_block(in_t[b], out_t[b], W, lanes, evecs)
                        _transpose_block(in_c[b], out_c[b], W, lanes, evecs)
                    pltpu.async_copy(out_t[b], st_hbm.at[pl.ds(row, E), :],
                                     sout_t[b])
                    pltpu.async_copy(out_c[b], sc_hbm.at[pl.ds(row, E), :],
                                     sout_c[b])
            return 0

        n_iter = (n_full + _NW - 1) // _NW + 1
        lax.fori_loop(0, (n_iter + 1) // 2, g_body, 0)
        for b in (0, 1):
            pltpu.make_async_copy(out_t[b], st_hbm.at[pl.ds(0, E), :],
                                  sout_t[b]).wait()
            pltpu.make_async_copy(out_c[b], sc_hbm.at[pl.ds(0, E), :],
                                  sout_c[b]).wait()

        if tail:
            @pl.when(wid < 2)
            def _():
                @pl.when(wid == 0)
                def _():
                    pltpu.sync_copy(tt_hbm.at[:, pl.ds(n_full * W, tail)],
                                    itail)
                    _transpose_block(itail, otail, tail, lanes, evecs)
                    pltpu.sync_copy(
                        otail, st_hbm.at[pl.ds(n_full * E, tail // 2), :])

                @pl.when(wid == 1)
                def _():
                    pltpu.sync_copy(tc_hbm.at[:, pl.ds(n_full * W, tail)],
                                    itail)
                    _transpose_block(itail, otail, tail, lanes, evecs)
                    pltpu.sync_copy(
                        otail, sc_hbm.at[pl.ds(n_full * E, tail // 2), :])

    return k1


def _make_dots_kernel(B, C, E, Cb):
    n_chunks = (B // _NW) // Cb
    assert Cb % _L == 0 and (B // _NW) % Cb == 0
    assert (Cb * C) % _GCHUNK == 0
    n_cgather = (Cb * C) // _GCHUNK
    W = 2 * E

    mesh = plsc.VectorSubcoreMesh(**_MESH)

    @functools.partial(
        pl.kernel,
        mesh=mesh,
        compiler_params=_PARAMS,
        out_type=jax.ShapeDtypeStruct((B * C,), jnp.float32),
        scratch_types=[
            pltpu.VMEM((Cb,), jnp.int32),
            pltpu.VMEM((Cb,), jnp.int32),
            pltpu.VMEM((Cb * C,), jnp.int32),
            pltpu.VMEM((Cb * C,), jnp.int32),
            pltpu.VMEM((Cb, W), jnp.float32),
            pltpu.VMEM((Cb * C, W), jnp.float32),
            pltpu.VMEM((Cb * C,), jnp.float32),
            pltpu.SemaphoreType.DMA,
        ],
    )
    def sc_k(thi_hbm, toff_hbm, chi_hbm, coff_hbm, ttab_hbm, ctab_hbm,
             out_hbm, tgt_idx, tgt_off, ctx_idx, ctx_off, te_rows, ce_rows,
             out_v, sem):
        wid = lax.axis_index("s") * _NC + lax.axis_index("c")
        lanes = lax.iota(jnp.int32, _L)

        for i in range(n_chunks):
            base_b = wid * (B // _NW) + i * Cb
            pltpu.sync_copy(thi_hbm.at[pl.ds(base_b, Cb)], tgt_idx)
            pltpu.sync_copy(toff_hbm.at[pl.ds(base_b, Cb)], tgt_off)
            pltpu.sync_copy(chi_hbm.at[pl.ds(base_b * C, Cb * C)], ctx_idx)
            pltpu.sync_copy(coff_hbm.at[pl.ds(base_b * C, Cb * C)], ctx_off)

            cps = [pltpu.async_copy(ttab_hbm.at[tgt_idx], te_rows, sem)]
            for j in range(n_cgather):
                cps.append(pltpu.async_copy(
                    ctab_hbm.at[ctx_idx.at[pl.ds(j * _GCHUNK, _GCHUNK)]],
                    ce_rows.at[pl.ds(j * _GCHUNK, _GCHUNK)], sem))
            for cp in cps:
                cp.wait()

            @plsc.parallel_loop(0, Cb // _L)
            def _(g):
                b_ids = g * _L + lanes
                flat0 = b_ids * C
                toff = plsc.load_gather(tgt_off, [b_ids])
                coffs = [plsc.load_gather(ctx_off, [flat0 + c])
                         for c in range(C)]
                zeros = tuple(jnp.zeros((_L,), jnp.float32)
                              for _ in range(C))

                @plsc.parallel_loop(0, E, unroll=2, carry=zeros)
                def accs(e, accs):
                    ev = (jnp.full((_L,), e, jnp.int32) + lanes) & (E - 1)
                    tv = plsc.load_gather(te_rows, [b_ids, toff + ev])
                    return tuple(
                        accs[c] + tv * plsc.load_gather(
                            ce_rows, [flat0 + c, coffs[c] + ev])
                        for c in range(C))

                for c in range(C):
                    plsc.store_scatter(out_v, [flat0 + c], accs[c])
            pltpu.sync_copy(out_v, out_hbm.at[pl.ds(base_b * C, Cb * C)])

    return sc_k


def kernel(target, context, target_table, context_table):
    B, C = context.shape
    V, E = target_table.shape
    ctx_flat = context.reshape(-1)
    t_hi = lax.shift_right_logical(target, 1)
    t_off = (target & 1) * E
    c_hi = lax.shift_right_logical(ctx_flat, 1)
    c_off = (ctx_flat & 1) * E
    k1 = _make_relayout_kernel(V, E)
    stag_t, stag_c = k1(target_table.T, context_table.T)
    sc_k = _make_dots_kernel(B, C, E, Cb=128)
    out = sc_k(t_hi, t_off, c_hi, c_off, stag_t, stag_c)
    return out.reshape(B, C)


# speedup vs baseline: 3.9100x; 1.5761x over previous
import functools

import jax
import jax.numpy as jnp
from jax import lax
from jax.experimental import pallas as pl
from jax.experimental.pallas import tpu as pltpu
from jax.experimental.pallas import tpu_sc as plsc

_NC = 2
_NS = 16
_L = 16
_NW = _NC * _NS

_GCHUNK = 128

_MESH = dict(core_axis_name="c", subcore_axis_name="s")
_SKIP_T = False
_PARAMS = pltpu.CompilerParams(needs_layout_passes=False)


def _transpose_block(in_ref, out_ref, n_cols, lanes, evecs):
    n_j = n_cols // 16

    @plsc.parallel_loop(0, 16, unroll=4)
    def _(m):
        cm = (jnp.full((_L,), m, jnp.int32) + lanes) & 15
        cmh = cm >> 1
        podd = (cm & 1) << 6
        ps = [evecs[q] + podd for q in range(4)]
        for j in range(n_j):
            cj = cm + (16 * j)
            rj = cmh + (8 * j)
            vs = [plsc.load_gather(in_ref, [evecs[q], cj]) for q in range(4)]
            for q in range(4):
                plsc.store_scatter(out_ref, [rj, ps[q]], vs[q])


def _make_relayout_kernel(V, E):
    W = 2 * E
    n_full = V // W
    tail = V - n_full * W
    r_out = V // 2

    mesh = plsc.VectorSubcoreMesh(**_MESH)

    @functools.partial(
        pl.kernel,
        mesh=mesh,
        compiler_params=_PARAMS,
        out_type=(jax.ShapeDtypeStruct((r_out, W), jnp.float32),
                  jax.ShapeDtypeStruct((r_out, W), jnp.float32)),
        scratch_types=[
            pltpu.VMEM((E, W), jnp.float32), pltpu.VMEM((E, W), jnp.float32),
            pltpu.VMEM((E, W), jnp.float32), pltpu.VMEM((E, W), jnp.float32),
            pltpu.VMEM((E, W), jnp.float32), pltpu.VMEM((E, W), jnp.float32),
            pltpu.VMEM((E, W), jnp.float32), pltpu.VMEM((E, W), jnp.float32),
            pltpu.VMEM((E, E), jnp.float32),
            pltpu.VMEM((E // 2, W), jnp.float32),
            pltpu.SemaphoreType.DMA, pltpu.SemaphoreType.DMA,
            pltpu.SemaphoreType.DMA, pltpu.SemaphoreType.DMA,
            pltpu.SemaphoreType.DMA, pltpu.SemaphoreType.DMA,
            pltpu.SemaphoreType.DMA, pltpu.SemaphoreType.DMA,
        ],
    )
    def k1(tt_hbm, tc_hbm, st_hbm, sc_hbm,
           it0, it1, ic0, ic1, ot0, ot1, oc0, oc1, itail, otail,
           sit0, sit1, sic0, sic1, sot0, sot1, soc0, soc1):
        in_t, in_c = (it0, it1), (ic0, ic1)
        out_t, out_c = (ot0, ot1), (oc0, oc1)
        sin_t, sin_c = (sit0, sit1), (sic0, sic1)
        sout_t, sout_c = (sot0, sot1), (soc0, soc1)

        wid = lax.axis_index("s") * _NC + lax.axis_index("c")
        lanes = lax.iota(jnp.int32, _L)
        evecs = [lanes + 16 * q for q in range(4)]

        def start_in(k, b):
            blk = wid + _NW * k

            @pl.when(blk < n_full)
            def _():
                col = blk * W
                pltpu.async_copy(tt_hbm.at[:, pl.ds(col, W)], in_t[b],
                                 sin_t[b])
                pltpu.async_copy(tc_hbm.at[:, pl.ds(col, W)], in_c[b],
                                 sin_c[b])

        start_in(0, 0)

        def g_body(g, _):
            for b in (0, 1):
                k = 2 * g + b
                start_in(k + 1, 1 - b)
                blk = wid + _NW * k

                @pl.when(blk < n_full)
                def _():
                    col = blk * W
                    row = blk * E
                    pltpu.make_async_copy(
                        tt_hbm.at[:, pl.ds(col, W)], in_t[b],
                        sin_t[b]).wait()
                    pltpu.make_async_copy(
                        tc_hbm.at[:, pl.ds(col, W)], in_c[b],
                        sin_c[b]).wait()

                    @pl.when(k >= 2)
                    def _():
                        pltpu.make_async_copy(
                            out_t[b], st_hbm.at[pl.ds(row, E), :],
                            sout_t[b]).wait()
                        pltpu.make_async_copy(
                            out_c[b], sc_hbm.at[pl.ds(row, E), :],
                            sout_c[b]).wait()

                    if _SKIP_T:
                        pass
                    else:
                        _transpose_block(in_t[b], out_t[b], W, lanes, evecs)
                        _transpose_block(in_c[b], out_c[b], W, lanes, evecs)
                    pltpu.async_copy(out_t[b], st_hbm.at[pl.ds(row, E), :],
                                     sout_t[b])
                    pltpu.async_copy(out_c[b], sc_hbm.at[pl.ds(row, E), :],
                                     sout_c[b])
            return 0

        n_iter = (n_full + _NW - 1) // _NW + 1
        lax.fori_loop(0, (n_iter + 1) // 2, g_body, 0)
        for b in (0, 1):
            pltpu.make_async_copy(out_t[b], st_hbm.at[pl.ds(0, E), :],
                                  sout_t[b]).wait()
            pltpu.make_async_copy(out_c[b], sc_hbm.at[pl.ds(0, E), :],
                                  sout_c[b]).wait()

        if tail:
            @pl.when(wid < 2)
            def _():
                @pl.when(wid == 0)
                def _():
                    pltpu.sync_copy(tt_hbm.at[:, pl.ds(n_full * W, tail)],
                                    itail)
                    _transpose_block(itail, otail, tail, lanes, evecs)
                    pltpu.sync_copy(
                        otail, st_hbm.at[pl.ds(n_full * E, tail // 2), :])

                @pl.when(wid == 1)
                def _():
                    pltpu.sync_copy(tc_hbm.at[:, pl.ds(n_full * W, tail)],
                                    itail)
                    _transpose_block(itail, otail, tail, lanes, evecs)
                    pltpu.sync_copy(
                        otail, sc_hbm.at[pl.ds(n_full * E, tail // 2), :])

    return k1


def _make_dots_kernel(B, C, E, Cb):
    n_chunks = (B // _NW) // Cb
    assert Cb % _L == 0 and (B // _NW) % Cb == 0
    assert (Cb * C) % _GCHUNK == 0
    n_cgather = (Cb * C) // _GCHUNK
    W = 2 * E

    mesh = plsc.VectorSubcoreMesh(**_MESH)

    @functools.partial(
        pl.kernel,
        mesh=mesh,
        compiler_params=_PARAMS,
        out_type=jax.ShapeDtypeStruct((B * C,), jnp.float32),
        scratch_types=[
            pltpu.VMEM((Cb,), jnp.int32),
            pltpu.VMEM((Cb,), jnp.int32),
            pltpu.VMEM((Cb * C,), jnp.int32),
            pltpu.VMEM((Cb * C,), jnp.int32),
            pltpu.VMEM((Cb, W), jnp.float32),
            pltpu.VMEM((Cb * C, W), jnp.float32),
            pltpu.VMEM((Cb * C,), jnp.float32),
            pltpu.SemaphoreType.DMA,
        ],
    )
    def sc_k(thi_hbm, toff_hbm, chi_hbm, coff_hbm, ttab_hbm, ctab_hbm,
             out_hbm, tgt_idx, tgt_off, ctx_idx, ctx_off, te_rows, ce_rows,
             out_v, sem):
        wid = lax.axis_index("s") * _NC + lax.axis_index("c")
        lanes = lax.iota(jnp.int32, _L)

        for i in range(n_chunks):
            base_b = wid * (B // _NW) + i * Cb
            pltpu.sync_copy(thi_hbm.at[pl.ds(base_b, Cb)], tgt_idx)
            pltpu.sync_copy(toff_hbm.at[pl.ds(base_b, Cb)], tgt_off)
            pltpu.sync_copy(chi_hbm.at[pl.ds(base_b * C, Cb * C)], ctx_idx)
            pltpu.sync_copy(coff_hbm.at[pl.ds(base_b * C, Cb * C)], ctx_off)

            cps = [pltpu.async_copy(ttab_hbm.at[tgt_idx], te_rows, sem)]
            for j in range(n_cgather):
                cps.append(pltpu.async_copy(
                    ctab_hbm.at[ctx_idx.at[pl.ds(j * _GCHUNK, _GCHUNK)]],
                    ce_rows.at[pl.ds(j * _GCHUNK, _GCHUNK)], sem))
            for cp in cps:
                cp.wait()

            @plsc.parallel_loop(0, Cb // _L)
            def _(g):
                b_ids = g * _L + lanes
                flat0 = b_ids * C
                toff = plsc.load_gather(tgt_off, [b_ids])
                coffs = [plsc.load_gather(ctx_off, [flat0 + c])
                         for c in range(C)]
                zeros = tuple(jnp.zeros((_L,), jnp.float32)
                              for _ in range(C))

                @plsc.parallel_loop(0, E, unroll=2, carry=zeros)
                def accs(e, accs):
                    ev = (jnp.full((_L,), e, jnp.int32) + lanes) & (E - 1)
                    tv = plsc.load_gather(te_rows, [b_ids, toff + ev])
                    return tuple(
                        accs[c] + tv * plsc.load_gather(
                            ce_rows, [flat0 + c, coffs[c] + ev])
                        for c in range(C))

                for c in range(C):
                    plsc.store_scatter(out_v, [flat0 + c], accs[c])
            pltpu.sync_copy(out_v, out_hbm.at[pl.ds(base_b * C, Cb * C)])

    return sc_k


def kernel(target, context, target_table, context_table):
    B, C = context.shape
    V, E = target_table.shape
    ctx_flat = context.reshape(-1)
    t_hi = lax.shift_right_logical(target, 1)
    t_off = (target & 1) * E
    c_hi = lax.shift_right_logical(ctx_flat, 1)
    c_off = (ctx_flat & 1) * E
    k1 = _make_relayout_kernel(V, E)
    stag_t, stag_c = k1(target_table.T, context_table.T)
    sc_k = _make_dots_kernel(B, C, E, Cb=128)
    out = sc_k(t_hi, t_off, c_hi, c_off, stag_t, stag_c)
    return out.reshape(B, C)
